# baseline (speedup 1.0000x reference)
import jax
import jax.numpy as jnp
from jax import lax
from jax.experimental import pallas as pl
from jax.experimental.pallas import tpu as pltpu
from jax.experimental.pallas import tpu_sc as plsc

_F = 26
_V = 100000
_E = 64
_B = 4096
_NC = 2
_NS = 16
_NW = _NC * _NS
_TOTAL = _B * _F
_PER_W = _TOTAL // _NW
_G = 16
_WPB = 8
_RING = 4
_FLUSH = _G * _WPB
_NBLK = _PER_W // _FLUSH


def _fire_wave(tab_hbm, row_v, p0, tbuf, sem):
    rvec = row_v[pl.ds(p0, _G)]
    for j in range(_G):
        row = rvec[j]
        r8 = pl.multiple_of((row >> 3) << 3, 8)
        pltpu.async_copy(
            tab_hbm.at[pl.ds(r8, 8), :], tbuf.at[pl.ds(j * 8, 8), :], sem
        )


def _drain_wave(tab_hbm, row_v, p0, tbuf, sem, obuf, o0):
    pltpu.make_async_copy(tab_hbm.at[pl.ds(0, _G * 8), :], tbuf, sem).wait()
    rvec = row_v[pl.ds(p0, _G)]
    lanes = lax.iota(jnp.int32, _G)
    rowvec = lanes * 8 + lax.rem(rvec, 8)
    dstvec = lanes * _E + o0
    col = lanes * 0
    for _e in range(_E):
        vals = plsc.load_gather(tbuf, [rowvec, col])
        plsc.store_scatter(obuf, [dstvec + _e], vals)
        col = col + 1


def _body(x_hbm, tab_hbm, out_hbm, idx_v, *rest):
    tbufs = rest[:_RING]
    ob0, ob1 = rest[_RING], rest[_RING + 1]
    sems = rest[_RING + 2 : 2 * _RING + 2]
    wsems = rest[2 * _RING + 2 : 2 * _RING + 4]
    obs = (ob0, ob1)

    wid = lax.axis_index("s") * _NC + lax.axis_index("c")
    base = wid * _PER_W
    pltpu.sync_copy(x_hbm.at[pl.ds(base, _PER_W)], idx_v.at[pl.ds(0, _PER_W)])

    def adjust(i, carry):
        p0 = i * _G
        vals = idx_v[pl.ds(p0, _G)]
        fld = lax.rem(lax.iota(jnp.int32, _G) + p0, _F)
        idx_v[pl.ds(p0, _G)] = vals + fld * _V
        return carry

    lax.fori_loop(0, _PER_W // _G, adjust, 0)

    for i in range(_FLUSH // _G):
        idx_v[pl.ds(_PER_W + i * _G, _G)] = jnp.zeros((_G,), jnp.int32)

    def do_block(p_blk, ob, guard):
        obuf = obs[ob]

        @pl.when(guard)
        def _wait_prev_flush():
            pltpu.make_async_copy(
                out_hbm.at[pl.ds(0, _FLUSH * _E)], obuf, wsems[ob]
            ).wait()

        for w in range(_WPB):
            slot = w % _RING
            _drain_wave(tab_hbm, idx_v, p_blk + w * _G, tbufs[slot], sems[slot], obuf, w * _G * _E)
            _fire_wave(tab_hbm, idx_v, p_blk + (w + _RING) * _G, tbufs[slot], sems[slot])
        pltpu.async_copy(
            obuf, out_hbm.at[pl.ds((base + p_blk) * _E, _FLUSH * _E)], wsems[ob]
        )

    for w in range(_RING):
        _fire_wave(tab_hbm, idx_v, w * _G, tbufs[w], sems[w])

    def pair(bp, carry):
        p = bp * 2 * _FLUSH
        do_block(p, 0, bp > 0)
        do_block(p + _FLUSH, 1, bp > 0)
        return carry

    lax.fori_loop(0, _NBLK // 2, pair, 0)

    for w in range(_RING):
        pltpu.make_async_copy(
            tab_hbm.at[pl.ds(0, _G * 8), :], tbufs[w], sems[w]
        ).wait()
    for ob in range(2):
        pltpu.make_async_copy(
            out_hbm.at[pl.ds(0, _FLUSH * _E)], obs[ob], wsems[ob]
        ).wait()


@jax.jit
def kernel(X, tables):
    xflat = X.reshape(_TOTAL)
    run = pl.kernel(
        _body,
        out_type=jax.ShapeDtypeStruct((_TOTAL * _E,), jnp.float32),
        mesh=plsc.VectorSubcoreMesh(core_axis_name="c", subcore_axis_name="s"),
        compiler_params=pltpu.CompilerParams(
            use_tc_tiling_on_sc=True, needs_layout_passes=False
        ),
        scratch_types=[pltpu.VMEM((_PER_W + _FLUSH,), jnp.int32)]
        + [pltpu.VMEM((_G * 8, _E), jnp.float32) for _ in range(_RING)]
        + [pltpu.VMEM((_FLUSH * _E,), jnp.float32) for _ in range(2)]
        + [pltpu.SemaphoreType.DMA for _ in range(_RING + 2)],
    )
    out = run(xflat, tables.reshape(_F * _V, _E))
    return out.reshape(_B, 1, _F * _E)


# speedup vs baseline: 1.1364x; 1.1364x over previous
import jax
import jax.numpy as jnp
from jax import lax
from jax.experimental import pallas as pl
from jax.experimental.pallas import tpu as pltpu
from jax.experimental.pallas import tpu_sc as plsc

_F = 26
_V = 100000
_E = 64
_B = 4096
_NC = 2
_NS = 16
_NW = _NC * _NS
_TOTAL = _B * _F
_PER_W = _TOTAL // _NW
_G = 16
_WPB = 8
_RING = 4
_FLUSH = _G * _WPB
_NBLK = _PER_W // _FLUSH


def _fire_wave(tab_hbm, row_v, p0, tbuf, sem):
    rvec = row_v[pl.ds(p0, _G)]
    for j in range(_G):
        row = rvec[j]
        r8 = pl.multiple_of((row >> 3) << 3, 8)
        pltpu.async_copy(
            tab_hbm.at[pl.ds(r8, 8), :], tbuf.at[pl.ds(j * 8, 8), :], sem
        )


def _drain_wave(tab_hbm, row_v, p0, tbuf, sem, obuf, o0):
    pltpu.make_async_copy(tab_hbm.at[pl.ds(0, _G * 8), :], tbuf, sem).wait()
    rvec = row_v[pl.ds(p0, _G)]
    for j in range(_G):
        r = j * 8 + lax.rem(rvec[j], 8)
        for k in range(_E // 16):
            obuf[pl.ds(o0 + j * _E + k * 16, 16)] = tbuf[r, pl.ds(k * 16, 16)]


def _body(x_hbm, tab_hbm, out_hbm, idx_v, *rest):
    tbufs = rest[:_RING]
    ob0, ob1 = rest[_RING], rest[_RING + 1]
    sems = rest[_RING + 2 : 2 * _RING + 2]
    wsems = rest[2 * _RING + 2 : 2 * _RING + 4]
    obs = (ob0, ob1)

    wid = lax.axis_index("s") * _NC + lax.axis_index("c")
    base = wid * _PER_W
    pltpu.sync_copy(x_hbm.at[pl.ds(base, _PER_W)], idx_v.at[pl.ds(0, _PER_W)])

    def adjust(i, carry):
        p0 = i * _G
        vals = idx_v[pl.ds(p0, _G)]
        fld = lax.rem(lax.iota(jnp.int32, _G) + p0, _F)
        idx_v[pl.ds(p0, _G)] = vals + fld * _V
        return carry

    lax.fori_loop(0, _PER_W // _G, adjust, 0)

    for i in range(_FLUSH // _G):
        idx_v[pl.ds(_PER_W + i * _G, _G)] = jnp.zeros((_G,), jnp.int32)

    def do_block(p_blk, ob, guard):
        obuf = obs[ob]

        @pl.when(guard)
        def _wait_prev_flush():
            pltpu.make_async_copy(
                out_hbm.at[pl.ds(0, _FLUSH * _E)], obuf, wsems[ob]
            ).wait()

        for w in range(_WPB):
            slot = w % _RING
            _drain_wave(tab_hbm, idx_v, p_blk + w * _G, tbufs[slot], sems[slot], obuf, w * _G * _E)
            _fire_wave(tab_hbm, idx_v, p_blk + (w + _RING) * _G, tbufs[slot], sems[slot])
        pltpu.async_copy(
            obuf, out_hbm.at[pl.ds((base + p_blk) * _E, _FLUSH * _E)], wsems[ob]
        )

    for w in range(_RING):
        _fire_wave(tab_hbm, idx_v, w * _G, tbufs[w], sems[w])

    def pair(bp, carry):
        p = bp * 2 * _FLUSH
        do_block(p, 0, bp > 0)
        do_block(p + _FLUSH, 1, bp > 0)
        return carry

    lax.fori_loop(0, _NBLK // 2, pair, 0)

    for w in range(_RING):
        pltpu.make_async_copy(
            tab_hbm.at[pl.ds(0, _G * 8), :], tbufs[w], sems[w]
        ).wait()
    for ob in range(2):
        pltpu.make_async_copy(
            out_hbm.at[pl.ds(0, _FLUSH * _E)], obs[ob], wsems[ob]
        ).wait()


@jax.jit
def kernel(X, tables):
    xflat = X.reshape(_TOTAL)
    run = pl.kernel(
        _body,
        out_type=jax.ShapeDtypeStruct((_TOTAL * _E,), jnp.float32),
        mesh=plsc.VectorSubcoreMesh(core_axis_name="c", subcore_axis_name="s"),
        compiler_params=pltpu.CompilerParams(use_tc_tiling_on_sc=True),
        scratch_types=[pltpu.VMEM((_PER_W + _FLUSH,), jnp.int32)]
        + [pltpu.VMEM((_G * 8, _E), jnp.float32) for _ in range(_RING)]
        + [pltpu.VMEM((_FLUSH * _E,), jnp.float32) for _ in range(2)]
        + [pltpu.SemaphoreType.DMA for _ in range(_RING + 2)],
    )
    out = run(xflat, tables.reshape(_F * _V, _E))
    return out.reshape(_B, 1, _F * _E)
